# baseline (speedup 1.0000x reference)
import functools

import jax
import jax.numpy as jnp
from jax.experimental import pallas as pl
from jax.experimental.pallas import tpu as pltpu

_HI = jax.lax.Precision.HIGHEST


def _mm(a, b, prec=None):
    return jax.lax.dot_general(a, b, (((1,), (0,)), ((), ())),
                               preferred_element_type=jnp.float32,
                               precision=prec)


def _mm_t(a, b, prec=None):
    return jax.lax.dot_general(a, b, (((1,), (1,)), ((), ())),
                               preferred_element_type=jnp.float32,
                               precision=prec)


def _fused(x_ref,
           eW0, eb0, eW1, eb1, eW2, eb2, eW3, eb3,
           dW0, db0, dW1, db1, dW2, db2, dW3, db3,
           cb_ref,
           out_ref, idx_ref, loss_ref,
           *, n_steps, n_rows, n_codebooks, n_codes, beta):
    step = pl.program_id(0)

    h = x_ref[...]
    for W, b in ((eW0, eb0), (eW1, eb1), (eW2, eb2)):
        h = jax.nn.relu(_mm(h, W[...]) + b[...])
    h = _mm(h, eW3[...]) + eb3[...]

    e_dim = h.shape[1]
    ones_row = jnp.ones((1, e_dim), dtype=jnp.float32)
    residual = h
    loss_sq = jnp.float32(0.0)
    idx_cols = []
    iota_f = jax.lax.broadcasted_iota(
        jnp.int32, (h.shape[0], n_codes), 1).astype(jnp.float32)
    for i in range(n_codebooks):
        cb = cb_ref[i]
        cbn2 = cb * jnp.float32(-2.0)
        cbsq = cb * cb
        cbsq_hi = cbsq.astype(jnp.bfloat16).astype(jnp.float32)
        csq = _mm_t(ones_row, cbsq_hi) + _mm_t(ones_row, cbsq - cbsq_hi)
        rsq = jnp.sum(residual * residual, axis=1, keepdims=True)
        d = rsq + _mm_t(residual, cbn2) + csq
        md = jnp.min(d, axis=1, keepdims=True)
        masked = jnp.where(d <= md, iota_f, jnp.float32(n_codes))
        idxf = jnp.min(masked, axis=1, keepdims=True)
        one_hot = (masked == idxf).astype(jnp.float32)
        idx = idxf.astype(jnp.int32)
        cb_hi = cb.astype(jnp.bfloat16).astype(jnp.float32)
        cb_lo = cb - cb_hi
        q = _mm(one_hot, cb_hi) + _mm(one_hot, cb_lo)
        residual = residual - q
        loss_sq = loss_sq + jnp.sum(residual * residual)
        idx_cols.append(idx)
    x_q = h - residual

    idx_ref[...] = jnp.concatenate(idx_cols, axis=1)

    scale = (1.0 + beta) / (n_codebooks * n_rows)

    @pl.when(step == 0)
    def _init():
        loss_ref[0, 0] = 0.0

    loss_ref[0, 0] += loss_sq * scale

    h = x_q
    for W, b in ((dW0, db0), (dW1, db1), (dW2, db2)):
        h = jax.nn.relu(_mm(h, W[...]) + b[...])
    out_ref[...] = _mm(h, dW3[...]) + db3[...]


def kernel(x, enc_W0, enc_b0, enc_W1, enc_b1, enc_W2, enc_b2, enc_W3, enc_b3,
           dec_W0, dec_b0, dec_W1, dec_b1, dec_W2, dec_b2, dec_W3, dec_b3,
           codebooks):
    B, in_dim = x.shape
    n_codebooks, n_codes, e_dim = codebooks.shape
    BB = 1024
    n_steps = B // BB

    biases = [b.reshape(1, -1) for b in
              (enc_b0, enc_b1, enc_b2, enc_b3, dec_b0, dec_b1, dec_b2, dec_b3)]
    eb0, eb1, eb2, eb3, db0, db1, db2, db3 = biases

    def _full(a):
        return pl.BlockSpec(a.shape, lambda i: (0,) * a.ndim)

    in_specs = [pl.BlockSpec((BB, in_dim), lambda i: (i, 0))]
    weights = (enc_W0, eb0, enc_W1, eb1, enc_W2, eb2, enc_W3, eb3,
               dec_W0, db0, dec_W1, db1, dec_W2, db2, dec_W3, db3, codebooks)
    in_specs += [_full(w) for w in weights]

    out_shapes = (
        jax.ShapeDtypeStruct((B, dec_W3.shape[1]), jnp.float32),
        jax.ShapeDtypeStruct((B, n_codebooks), jnp.int32),
        jax.ShapeDtypeStruct((1, 1), jnp.float32),
    )
    out_specs = (
        pl.BlockSpec((BB, dec_W3.shape[1]), lambda i: (i, 0)),
        pl.BlockSpec((BB, n_codebooks), lambda i: (i, 0)),
        pl.BlockSpec((1, 1), lambda i: (0, 0), memory_space=pltpu.SMEM),
    )

    body = functools.partial(_fused, n_steps=n_steps, n_rows=B,
                             n_codebooks=n_codebooks, n_codes=n_codes,
                             beta=0.25)
    out, indices, loss = pl.pallas_call(
        body,
        grid=(n_steps,),
        in_specs=in_specs,
        out_specs=out_specs,
        out_shape=out_shapes,
        compiler_params=pltpu.CompilerParams(
            dimension_semantics=("arbitrary",)),
    )(x, *weights)

    return out, loss[0, 0], indices


# speedup vs baseline: 1.1182x; 1.1182x over previous
import functools

import jax
import jax.numpy as jnp
from jax.experimental import pallas as pl
from jax.experimental.pallas import tpu as pltpu

_HI = jax.lax.Precision.HIGHEST


def _mm(a, b, prec=None):
    return jax.lax.dot_general(a, b, (((1,), (0,)), ((), ())),
                               preferred_element_type=jnp.float32,
                               precision=prec)


def _mm_t(a, b, prec=None):
    return jax.lax.dot_general(a, b, (((1,), (1,)), ((), ())),
                               preferred_element_type=jnp.float32,
                               precision=prec)


def _fused(x_ref,
           eW0, eb0, eW1, eb1, eW2, eb2, eW3, eb3,
           dW0, db0, dW1, db1, dW2, db2, dW3, db3,
           cb_ref,
           out_ref, idx_ref, loss_ref,
           *, n_steps, n_rows, n_codebooks, n_codes, beta):
    step = pl.program_id(0)

    h = x_ref[...]
    for W, b in ((eW0, eb0), (eW1, eb1), (eW2, eb2)):
        h = jax.nn.relu(_mm(h, W[...]) + b[...])
    h = _mm(h, eW3[...]) + eb3[...]

    e_dim = h.shape[1]
    half = h.shape[0] // 2
    ones_row = jnp.ones((1, e_dim), dtype=jnp.float32)
    loss_sq = jnp.float32(0.0)
    iota_f = jax.lax.broadcasted_iota(
        jnp.int32, (half, n_codes), 1).astype(jnp.float32)
    res = [h[:half], h[half:]]
    idx_cols = [[], []]
    for i in range(n_codebooks):
        cb = cb_ref[i]
        cbn2 = cb * jnp.float32(-2.0)
        cbsq = cb * cb
        cbsq_hi = cbsq.astype(jnp.bfloat16).astype(jnp.float32)
        csq = _mm_t(ones_row, cbsq_hi) + _mm_t(ones_row, cbsq - cbsq_hi)
        cb_hi = cb.astype(jnp.bfloat16).astype(jnp.float32)
        cb_lo = cb - cb_hi
        d = [None, None]
        for p in range(2):
            rsq = jnp.sum(res[p] * res[p], axis=1, keepdims=True)
            d[p] = rsq + _mm_t(res[p], cbn2) + csq
        for p in range(2):
            md = jnp.min(d[p], axis=1, keepdims=True)
            masked = jnp.where(d[p] <= md, iota_f, jnp.float32(n_codes))
            idxf = jnp.min(masked, axis=1, keepdims=True)
            one_hot = (masked == idxf).astype(jnp.float32)
            idx_cols[p].append(idxf.astype(jnp.int32))
            q = _mm(one_hot, cb_hi) + _mm(one_hot, cb_lo)
            res[p] = res[p] - q
            loss_sq = loss_sq + jnp.sum(res[p] * res[p])
    x_q = jnp.concatenate([h[:half] - res[0], h[half:] - res[1]], axis=0)

    idx_ref[:half, :] = jnp.concatenate(idx_cols[0], axis=1)
    idx_ref[half:, :] = jnp.concatenate(idx_cols[1], axis=1)

    scale = (1.0 + beta) / (n_codebooks * n_rows)

    @pl.when(step == 0)
    def _init():
        loss_ref[0, 0] = 0.0

    loss_ref[0, 0] += loss_sq * scale

    h = x_q
    for W, b in ((dW0, db0), (dW1, db1), (dW2, db2)):
        h = jax.nn.relu(_mm(h, W[...]) + b[...])
    out_ref[...] = _mm(h, dW3[...]) + db3[...]


def kernel(x, enc_W0, enc_b0, enc_W1, enc_b1, enc_W2, enc_b2, enc_W3, enc_b3,
           dec_W0, dec_b0, dec_W1, dec_b1, dec_W2, dec_b2, dec_W3, dec_b3,
           codebooks):
    B, in_dim = x.shape
    n_codebooks, n_codes, e_dim = codebooks.shape
    BB = 2048
    n_steps = B // BB

    biases = [b.reshape(1, -1) for b in
              (enc_b0, enc_b1, enc_b2, enc_b3, dec_b0, dec_b1, dec_b2, dec_b3)]
    eb0, eb1, eb2, eb3, db0, db1, db2, db3 = biases

    def _full(a):
        return pl.BlockSpec(a.shape, lambda i: (0,) * a.ndim)

    in_specs = [pl.BlockSpec((BB, in_dim), lambda i: (i, 0))]
    weights = (enc_W0, eb0, enc_W1, eb1, enc_W2, eb2, enc_W3, eb3,
               dec_W0, db0, dec_W1, db1, dec_W2, db2, dec_W3, db3, codebooks)
    in_specs += [_full(w) for w in weights]

    out_shapes = (
        jax.ShapeDtypeStruct((B, dec_W3.shape[1]), jnp.float32),
        jax.ShapeDtypeStruct((B, n_codebooks), jnp.int32),
        jax.ShapeDtypeStruct((1, 1), jnp.float32),
    )
    out_specs = (
        pl.BlockSpec((BB, dec_W3.shape[1]), lambda i: (i, 0)),
        pl.BlockSpec((BB, n_codebooks), lambda i: (i, 0)),
        pl.BlockSpec((1, 1), lambda i: (0, 0), memory_space=pltpu.SMEM),
    )

    body = functools.partial(_fused, n_steps=n_steps, n_rows=B,
                             n_codebooks=n_codebooks, n_codes=n_codes,
                             beta=0.25)
    out, indices, loss = pl.pallas_call(
        body,
        grid=(n_steps,),
        in_specs=in_specs,
        out_specs=out_specs,
        out_shape=out_shapes,
        compiler_params=pltpu.CompilerParams(
            dimension_semantics=("arbitrary",)),
    )(x, *weights)

    return out, loss[0, 0], indices


# speedup vs baseline: 1.1283x; 1.0091x over previous
import functools

import jax
import jax.numpy as jnp
from jax.experimental import pallas as pl
from jax.experimental.pallas import tpu as pltpu

_HI = jax.lax.Precision.HIGHEST


def _mm(a, b, prec=None):
    return jax.lax.dot_general(a, b, (((1,), (0,)), ((), ())),
                               preferred_element_type=jnp.float32,
                               precision=prec)


def _mm_t(a, b, prec=None):
    return jax.lax.dot_general(a, b, (((1,), (1,)), ((), ())),
                               preferred_element_type=jnp.float32,
                               precision=prec)


def _fused(x_ref,
           eW0, eb0, eW1, eb1, eW2, eb2, eW3, eb3,
           dW0, db0, dW1, db1, dW2, db2, dW3, db3,
           cb_ref,
           out_ref, idx_ref, loss_ref,
           *, n_steps, n_rows, n_codebooks, n_codes, beta):
    step = pl.program_id(0)

    h = x_ref[...]
    for W, b in ((eW0, eb0), (eW1, eb1), (eW2, eb2)):
        h = jax.nn.relu(_mm(h, W[...]) + b[...])
    h = _mm(h, eW3[...]) + eb3[...]

    e_dim = h.shape[1]
    half = h.shape[0] // 4
    ones_row = jnp.ones((1, e_dim), dtype=jnp.float32)
    loss_sq = jnp.float32(0.0)
    iota_f = jax.lax.broadcasted_iota(
        jnp.int32, (half, n_codes), 1).astype(jnp.float32)
    res = [h[j * half:(j + 1) * half] for j in range(4)]
    idx_cols = [[], [], [], []]
    for i in range(n_codebooks):
        cb = cb_ref[i]
        cbn2 = cb * jnp.float32(-2.0)
        cbsq = cb * cb
        cbsq_hi = cbsq.astype(jnp.bfloat16).astype(jnp.float32)
        csq = _mm_t(ones_row, cbsq_hi) + _mm_t(ones_row, cbsq - cbsq_hi)
        cb_hi = cb.astype(jnp.bfloat16).astype(jnp.float32)
        cb_lo = cb - cb_hi
        d = [None] * 4
        for p in range(4):
            rsq = jnp.sum(res[p] * res[p], axis=1, keepdims=True)
            d[p] = rsq + _mm_t(res[p], cbn2) + csq
        for p in range(4):
            md = jnp.min(d[p], axis=1, keepdims=True)
            masked = jnp.where(d[p] <= md, iota_f, jnp.float32(n_codes))
            idxf = jnp.min(masked, axis=1, keepdims=True)
            one_hot = (masked == idxf).astype(jnp.float32)
            idx_cols[p].append(idxf.astype(jnp.int32))
            q = _mm(one_hot, cb_hi) + _mm(one_hot, cb_lo)
            res[p] = res[p] - q
            loss_sq = loss_sq + jnp.sum(res[p] * res[p])
    x_q = jnp.concatenate([h[j * half:(j + 1) * half] - res[j] for j in range(4)], axis=0)

    for j in range(4):
        idx_ref[pl.ds(j * half, half), :] = jnp.concatenate(idx_cols[j], axis=1)

    scale = (1.0 + beta) / (n_codebooks * n_rows)

    @pl.when(step == 0)
    def _init():
        loss_ref[0, 0] = 0.0

    loss_ref[0, 0] += loss_sq * scale

    h = x_q
    for W, b in ((dW0, db0), (dW1, db1), (dW2, db2)):
        h = jax.nn.relu(_mm(h, W[...]) + b[...])
    out_ref[...] = _mm(h, dW3[...]) + db3[...]


def kernel(x, enc_W0, enc_b0, enc_W1, enc_b1, enc_W2, enc_b2, enc_W3, enc_b3,
           dec_W0, dec_b0, dec_W1, dec_b1, dec_W2, dec_b2, dec_W3, dec_b3,
           codebooks):
    B, in_dim = x.shape
    n_codebooks, n_codes, e_dim = codebooks.shape
    BB = 2048
    n_steps = B // BB

    biases = [b.reshape(1, -1) for b in
              (enc_b0, enc_b1, enc_b2, enc_b3, dec_b0, dec_b1, dec_b2, dec_b3)]
    eb0, eb1, eb2, eb3, db0, db1, db2, db3 = biases

    def _full(a):
        return pl.BlockSpec(a.shape, lambda i: (0,) * a.ndim)

    in_specs = [pl.BlockSpec((BB, in_dim), lambda i: (i, 0))]
    weights = (enc_W0, eb0, enc_W1, eb1, enc_W2, eb2, enc_W3, eb3,
               dec_W0, db0, dec_W1, db1, dec_W2, db2, dec_W3, db3, codebooks)
    in_specs += [_full(w) for w in weights]

    out_shapes = (
        jax.ShapeDtypeStruct((B, dec_W3.shape[1]), jnp.float32),
        jax.ShapeDtypeStruct((B, n_codebooks), jnp.int32),
        jax.ShapeDtypeStruct((1, 1), jnp.float32),
    )
    out_specs = (
        pl.BlockSpec((BB, dec_W3.shape[1]), lambda i: (i, 0)),
        pl.BlockSpec((BB, n_codebooks), lambda i: (i, 0)),
        pl.BlockSpec((1, 1), lambda i: (0, 0), memory_space=pltpu.SMEM),
    )

    body = functools.partial(_fused, n_steps=n_steps, n_rows=B,
                             n_codebooks=n_codebooks, n_codes=n_codes,
                             beta=0.25)
    out, indices, loss = pl.pallas_call(
        body,
        grid=(n_steps,),
        in_specs=in_specs,
        out_specs=out_specs,
        out_shape=out_shapes,
        compiler_params=pltpu.CompilerParams(
            dimension_semantics=("arbitrary",)),
    )(x, *weights)

    return out, loss[0, 0], indices
